# baseline (speedup 1.0000x reference)
import functools

import jax
import jax.numpy as jnp
from jax import lax
from jax.experimental import pallas as pl
from jax.experimental.pallas import tpu as pltpu
from jax.experimental.pallas import tpu_sc as plsc

N_PAIRS = 8000
NHID = 128
NC = 2
NS = 16
NW = NC * NS
PPW = 256
LAST_BASE = (NW - 1) * PPW
LAST_CNT = N_PAIRS - LAST_BASE
FCH = NHID // 16


def _lane_gather(x, idx):
    dn = lax.GatherDimensionNumbers(
        offset_dims=(), collapsed_slice_dims=(0,), start_index_map=(0,))
    return lax.gather(x, idx[:, None], dn, slice_sizes=(1,),
                      mode=lax.GatherScatterMode.PROMISE_IN_BOUNDS)


def _sc_body(emb_h, np_h, w_h, out_h,
             idx0_v, idx1_v, w_v, r0_v, r1_v, s_v,
             semi0, semi1, semr0, semr1):
    cid = lax.axis_index("c")
    sid = lax.axis_index("s")
    wid = sid * NC + cid

    lane = lax.iota(jnp.int32, 16)
    perms = [jnp.bitwise_xor(lane, jnp.int32(k)) for k in (8, 4, 2, 1)]

    NPH = 4

    def run(base, cnt, sems):
        ph = cnt // NPH
        icps = []
        nic = 1 if cnt >= 128 else 2
        for k in range(nic):
            c = cnt // nic
            sli = pl.ds(k * c, c)
            hbi = pl.ds(base + k * c, c)
            icps.append(pltpu.async_copy(np_h.at[0, hbi], idx0_v.at[sli], sems[0]))
            icps.append(pltpu.async_copy(np_h.at[1, hbi], idx1_v.at[sli], sems[0]))
        for cp in icps:
            cp.wait()
        row_cp = []
        for f in range(NPH):
            sl = pl.ds(f * ph, ph)
            row_cp.append((
                pltpu.async_copy(emb_h.at[idx0_v.at[sl]], r0_v.at[sl], sems[f]),
                pltpu.async_copy(emb_h.at[idx1_v.at[sl]], r1_v.at[sl], sems[f]),
            ))

        i_even = (2 * lane) % 16
        i_odd = (2 * lane + 1) % 16
        lo8 = lane < 8
        wcs = []
        for ch in range(FCH):
            a = w_v[pl.ds(ch * 32, 16)]
            bvec = w_v[pl.ds(ch * 32 + 16, 16)]
            c0 = jnp.where(lo8, _lane_gather(a, i_even), _lane_gather(bvec, i_even))
            c1 = jnp.where(lo8, _lane_gather(a, i_odd), _lane_gather(bvec, i_odd))
            wcs.append(c1 - c0)

        def group(g, carry, off):
            svec = jnp.zeros((16,), jnp.float32)
            for j in range(16):
                p = off + g * 16 + j
                acc = jnp.zeros((16,), jnp.float32)
                for ch in range(FCH):
                    fsl = pl.ds(ch * 16, 16)
                    a = r0_v[p, fsl]
                    bb = r1_v[p, fsl]
                    acc = acc + jnp.abs(a - bb) * wcs[ch]
                for pm in perms:
                    acc = acc + _lane_gather(acc, pm)
                svec = jnp.where(lane == j, acc, svec)
            s_v[pl.ds(off + g * 16, 16)] = svec
            return carry

        for f in range(NPH):
            row_cp[f][0].wait()
            row_cp[f][1].wait()
            lax.fori_loop(0, ph // 16,
                          functools.partial(group, off=f * ph), 0)

    pltpu.sync_copy(w_h, w_v)

    @pl.when(wid < NW - 1)
    def _():
        base = wid * PPW
        run(base, PPW, (semi0, semi1, semr0, semr1))
        pltpu.sync_copy(s_v, out_h.at[pl.ds(base, PPW)])

    @pl.when(wid == NW - 1)
    def _():
        run(LAST_BASE, LAST_CNT, (semi0, semi1, semr0, semr1))
        pltpu.sync_copy(s_v.at[pl.ds(0, LAST_CNT)],
                        out_h.at[pl.ds(LAST_BASE, LAST_CNT)])


def _sc_dots(emb, npairs, w):
    mesh = plsc.VectorSubcoreMesh(core_axis_name="c", subcore_axis_name="s")
    fn = pl.kernel(
        _sc_body,
        mesh=mesh,
        out_type=jax.ShapeDtypeStruct((N_PAIRS,), jnp.float32),
        scratch_types=[
            pltpu.VMEM((PPW,), jnp.int32),
            pltpu.VMEM((PPW,), jnp.int32),
            pltpu.VMEM((NHID * 2,), jnp.float32),
            pltpu.VMEM((PPW, NHID), jnp.float32),
            pltpu.VMEM((PPW, NHID), jnp.float32),
            pltpu.VMEM((PPW,), jnp.float32),
            pltpu.SemaphoreType.DMA,
            pltpu.SemaphoreType.DMA,
            pltpu.SemaphoreType.DMA,
            pltpu.SemaphoreType.DMA,
        ],
    )
    return fn(emb, npairs, w)


def _tc_body(s_ref, lab_ref, b_ref, out_ref):
    s = s_ref[:]
    lab = lab_ref[:]
    sgn = jnp.where(lab == 0, jnp.float32(1.0), jnp.float32(-1.0))
    t = sgn * (s + (b_ref[1] - b_ref[0]))
    sp = jnp.maximum(t, 0.0) + jnp.log1p(jnp.exp(-jnp.abs(t)))
    out_ref[0, 0] = jnp.sum(sp) / N_PAIRS


def _tc_loss(s, lab, b):
    return pl.pallas_call(
        _tc_body,
        out_shape=jax.ShapeDtypeStruct((1, 1), jnp.float32),
        in_specs=[
            pl.BlockSpec(memory_space=pltpu.VMEM),
            pl.BlockSpec(memory_space=pltpu.VMEM),
            pl.BlockSpec(memory_space=pltpu.SMEM),
        ],
        out_specs=pl.BlockSpec(memory_space=pltpu.SMEM),
    )(s, lab, b)


def kernel(embeddings, node_pairs, labels, W, b):
    npairs = node_pairs.astype(jnp.int32)
    s = _sc_dots(embeddings, npairs, W.reshape(NHID * 2))
    loss = _tc_loss(s, labels.astype(jnp.int32), b)
    return loss[0, 0]


# speedup vs baseline: 1.1653x; 1.1653x over previous
import functools

import jax
import jax.numpy as jnp
from jax import lax
from jax.experimental import pallas as pl
from jax.experimental.pallas import tpu as pltpu
from jax.experimental.pallas import tpu_sc as plsc

N_PAIRS = 8000
NHID = 128
NC = 2
NS = 16
NW = NC * NS
PPW = 256
LAST_BASE = (NW - 1) * PPW
LAST_CNT = N_PAIRS - LAST_BASE
FCH = NHID // 16


def _lane_gather(x, idx):
    dn = lax.GatherDimensionNumbers(
        offset_dims=(), collapsed_slice_dims=(0,), start_index_map=(0,))
    return lax.gather(x, idx[:, None], dn, slice_sizes=(1,),
                      mode=lax.GatherScatterMode.PROMISE_IN_BOUNDS)


def _sc_body(emb_h, np_h, w_h, out_h,
             idx0_v, idx1_v, w_v, r0_v, r1_v, s_v,
             semi0, semi1, semr0, semr1):
    cid = lax.axis_index("c")
    sid = lax.axis_index("s")
    wid = sid * NC + cid

    lane = lax.iota(jnp.int32, 16)
    perms = [jnp.bitwise_xor(lane, jnp.int32(k)) for k in (8, 4, 2, 1)]

    def run(base, cnt, sems):
        ph = cnt // 2
        semi = [sems[0], sems[1]]
        semr = [sems[2], sems[3]]
        w_cp = pltpu.async_copy(w_h, w_v, semi[1])
        idx_cp = []
        nic = 1 if cnt >= 128 else 2
        for k in range(nic):
            c = cnt // nic
            sl = pl.ds(k * c, c)
            hb = pl.ds(base + k * c, c)
            idx_cp.append(pltpu.async_copy(np_h.at[0, hb], idx0_v.at[sl], semi[0]))
            idx_cp.append(pltpu.async_copy(np_h.at[1, hb], idx1_v.at[sl], semi[0]))
        for cp in idx_cp:
            cp.wait()
        row_cp = [None, None]
        sl0 = pl.ds(0, ph)
        row_cp[0] = (
            pltpu.async_copy(emb_h.at[idx0_v.at[sl0]], r0_v.at[sl0], semr[0]),
            pltpu.async_copy(emb_h.at[idx1_v.at[sl0]], r1_v.at[sl0], semr[0]),
        )
        sl1 = pl.ds(ph, ph)
        row_cp[1] = (
            pltpu.async_copy(emb_h.at[idx0_v.at[sl1]], r0_v.at[sl1], semr[1]),
            pltpu.async_copy(emb_h.at[idx1_v.at[sl1]], r1_v.at[sl1], semr[1]),
        )
        w_cp.wait()

        i_even = (2 * lane) % 16
        i_odd = (2 * lane + 1) % 16
        lo8 = lane < 8
        wcs = []
        for ch in range(FCH):
            a = w_v[pl.ds(ch * 32, 16)]
            bvec = w_v[pl.ds(ch * 32 + 16, 16)]
            c0 = jnp.where(lo8, _lane_gather(a, i_even), _lane_gather(bvec, i_even))
            c1 = jnp.where(lo8, _lane_gather(a, i_odd), _lane_gather(bvec, i_odd))
            wcs.append(c1 - c0)

        def group(g, carry, off):
            svec = jnp.zeros((16,), jnp.float32)
            for j in range(16):
                p = off + g * 16 + j
                acc = jnp.zeros((16,), jnp.float32)
                for ch in range(FCH):
                    fsl = pl.ds(ch * 16, 16)
                    a = r0_v[p, fsl]
                    bb = r1_v[p, fsl]
                    acc = acc + jnp.abs(a - bb) * wcs[ch]
                for pm in perms:
                    acc = acc + _lane_gather(acc, pm)
                svec = jnp.where(lane == j, acc, svec)
            s_v[pl.ds(off + g * 16, 16)] = svec
            return carry

        for f in range(2):
            row_cp[f][0].wait()
            row_cp[f][1].wait()
            lax.fori_loop(0, ph // 16,
                          functools.partial(group, off=f * ph), 0)

    @pl.when(wid < NW - 1)
    def _():
        base = wid * PPW
        run(base, PPW, (semi0, semi1, semr0, semr1))
        pltpu.sync_copy(s_v, out_h.at[pl.ds(base, PPW)])

    @pl.when(wid == NW - 1)
    def _():
        run(LAST_BASE, LAST_CNT, (semi0, semi1, semr0, semr1))
        pltpu.sync_copy(s_v.at[pl.ds(0, LAST_CNT)],
                        out_h.at[pl.ds(LAST_BASE, LAST_CNT)])


def _sc_dots(emb, npairs, w):
    mesh = plsc.VectorSubcoreMesh(core_axis_name="c", subcore_axis_name="s")
    fn = pl.kernel(
        _sc_body,
        mesh=mesh,
        out_type=jax.ShapeDtypeStruct((N_PAIRS,), jnp.float32),
        scratch_types=[
            pltpu.VMEM((PPW,), jnp.int32),
            pltpu.VMEM((PPW,), jnp.int32),
            pltpu.VMEM((NHID * 2,), jnp.float32),
            pltpu.VMEM((PPW, NHID), jnp.float32),
            pltpu.VMEM((PPW, NHID), jnp.float32),
            pltpu.VMEM((PPW,), jnp.float32),
            pltpu.SemaphoreType.DMA,
            pltpu.SemaphoreType.DMA,
            pltpu.SemaphoreType.DMA,
            pltpu.SemaphoreType.DMA,
        ],
    )
    return fn(emb, npairs, w)


def _tc_body(s_ref, lab_ref, b_ref, out_ref):
    s = s_ref[:]
    lab = lab_ref[:]
    sgn = jnp.where(lab == 0, jnp.float32(1.0), jnp.float32(-1.0))
    t = sgn * (s + (b_ref[1] - b_ref[0]))
    sp = jnp.maximum(t, 0.0) + jnp.log1p(jnp.exp(-jnp.abs(t)))
    out_ref[0, 0] = jnp.sum(sp) / N_PAIRS


def _tc_loss(s, lab, b):
    return pl.pallas_call(
        _tc_body,
        out_shape=jax.ShapeDtypeStruct((1, 1), jnp.float32),
        in_specs=[
            pl.BlockSpec(memory_space=pltpu.VMEM),
            pl.BlockSpec(memory_space=pltpu.VMEM),
            pl.BlockSpec(memory_space=pltpu.SMEM),
        ],
        out_specs=pl.BlockSpec(memory_space=pltpu.SMEM),
    )(s, lab, b)


def kernel(embeddings, node_pairs, labels, W, b):
    npairs = node_pairs.astype(jnp.int32)
    s = _sc_dots(embeddings, npairs, W.reshape(NHID * 2))
    loss = _tc_loss(s, labels.astype(jnp.int32), b)
    return loss[0, 0]
